# baseline (device time: 37334 ns/iter reference)
import jax
import jax.numpy as jnp
from jax import lax
from jax.experimental import pallas as pl
from jax.experimental.pallas import tpu as pltpu

N_DEV = 8
BLK = 2048


def kernel(x):
    x = x.astype(jnp.float32)
    m, n = x.shape
    nblk = m // BLK

    def body(
        x_ref, o_ref, carry_ref, comm_ref, stage_ref, send_sems, recv_sems
    ):
        b = pl.program_id(0)
        i = lax.axis_index("i")

        sends = [
            pltpu.make_async_remote_copy(
                src_ref=stage_ref,
                dst_ref=comm_ref.at[d - 1],
                send_sem=send_sems.at[d - 1],
                recv_sem=recv_sems.at[d - 1],
                device_id=(i + d,),
                device_id_type=pl.DeviceIdType.MESH,
            )
            for d in range(1, N_DEV)
        ]

        @pl.when(b == 0)
        def _():
            carry_ref[...] = jnp.ones_like(carry_ref)
            barrier_sem = pltpu.get_barrier_semaphore()
            for dd in range(1, N_DEV):
                pl.semaphore_signal(
                    barrier_sem, inc=1,
                    device_id=((i + dd) % N_DEV,),
                    device_id_type=pl.DeviceIdType.MESH,
                )
            pl.semaphore_wait(barrier_sem, N_DEV - 1)

        @pl.when(b == nblk - 1)
        def _():
            t = x_ref[...]
            r = BLK
            while r > 1:
                r //= 2
                t = t[:r, :] * t[r : 2 * r, :]
            stage_ref[...] = t * carry_ref[...]
            for d in range(1, N_DEV):
                @pl.when(i + d < N_DEV)
                def _():
                    sends[d - 1].start()

        y = x_ref[...]
        y = jnp.concatenate([y[0:1, :] * carry_ref[...], y[1:, :]], axis=0)
        s = 1
        while s < BLK:
            y = jnp.concatenate(
                [y[:s, :], y[s:, :] * y[: BLK - s, :]], axis=0
            )
            s *= 2
        o_ref[pl.ds(b * BLK, BLK), :] = y.astype(jnp.bfloat16)
        carry_ref[...] = y[BLK - 1 : BLK, :]

        @pl.when(b == nblk - 1)
        def _():
            for d in range(1, N_DEV):
                @pl.when(i >= d)
                def _():
                    sends[d - 1].wait_recv()

            @pl.when(i > 0)
            def _():
                p = jnp.ones((1, n), jnp.float32)
                for d in range(1, N_DEV):
                    p = p * jnp.where(i >= d, comm_ref[d - 1], 1.0)
                o_ref[...] = o_ref[...] * p.astype(jnp.bfloat16)

            for d in range(1, N_DEV):
                @pl.when(i + d < N_DEV)
                def _():
                    sends[d - 1].wait_send()

    return pl.pallas_call(
        body,
        grid=(nblk,),
        out_shape=jax.ShapeDtypeStruct((m, n), jnp.bfloat16),
        in_specs=[pl.BlockSpec((BLK, n), lambda b: (b, 0))],
        out_specs=pl.BlockSpec((m, n), lambda b: (0, 0)),
        scratch_shapes=[
            pltpu.VMEM((1, n), jnp.float32),
            pltpu.VMEM((N_DEV - 1, 1, n), jnp.float32),
            pltpu.VMEM((1, n), jnp.float32),
            pltpu.SemaphoreType.DMA((N_DEV - 1,)),
            pltpu.SemaphoreType.DMA((N_DEV - 1,)),
        ],
        compiler_params=pltpu.CompilerParams(collective_id=0),
    )(x)


# device time: 34820 ns/iter; 1.0722x vs baseline; 1.0722x over previous
import jax
import jax.numpy as jnp
from jax import lax
from jax.experimental import pallas as pl
from jax.experimental.pallas import tpu as pltpu

N_DEV = 8
BLK = 1024


def kernel(x):
    x = x.astype(jnp.float32)
    m, n = x.shape
    nblk = m // BLK

    def body(
        x_ref, o_ref, carry_ref, comm_ref, stage_ref, send_sems, recv_sems
    ):
        b = pl.program_id(0)
        i = lax.axis_index("i")

        sends = [
            pltpu.make_async_remote_copy(
                src_ref=stage_ref,
                dst_ref=comm_ref.at[d - 1],
                send_sem=send_sems.at[d - 1],
                recv_sem=recv_sems.at[d - 1],
                device_id=(i + d,),
                device_id_type=pl.DeviceIdType.MESH,
            )
            for d in range(1, N_DEV)
        ]

        @pl.when(b == 0)
        def _():
            carry_ref[...] = jnp.ones_like(carry_ref)
            barrier_sem = pltpu.get_barrier_semaphore()
            for dd in range(1, N_DEV):
                pl.semaphore_signal(
                    barrier_sem, inc=1,
                    device_id=((i + dd) % N_DEV,),
                    device_id_type=pl.DeviceIdType.MESH,
                )
            pl.semaphore_wait(barrier_sem, N_DEV - 1)

        @pl.when(b == nblk - 1)
        def _():
            t = x_ref[...]
            r = BLK
            while r > 1:
                r //= 2
                t = t[:r, :] * t[r : 2 * r, :]
            stage_ref[...] = t * carry_ref[...]
            for d in range(1, N_DEV):
                @pl.when(i + d < N_DEV)
                def _():
                    sends[d - 1].start()

        y = x_ref[...]
        y = jnp.concatenate([y[0:1, :] * carry_ref[...], y[1:, :]], axis=0)
        s = 1
        while s < BLK:
            y = jnp.concatenate(
                [y[:s, :], y[s:, :] * y[: BLK - s, :]], axis=0
            )
            s *= 2
        o_ref[pl.ds(b * BLK, BLK), :] = y.astype(jnp.bfloat16)
        carry_ref[...] = y[BLK - 1 : BLK, :]

        @pl.when(b == nblk - 1)
        def _():
            for d in range(1, N_DEV):
                @pl.when(i >= d)
                def _():
                    sends[d - 1].wait_recv()

            @pl.when(i > 0)
            def _():
                p = jnp.ones((1, n), jnp.float32)
                for d in range(1, N_DEV):
                    p = p * jnp.where(i >= d, comm_ref[d - 1], 1.0)
                o_ref[...] = o_ref[...] * p.astype(jnp.bfloat16)

            for d in range(1, N_DEV):
                @pl.when(i + d < N_DEV)
                def _():
                    sends[d - 1].wait_send()

    return pl.pallas_call(
        body,
        grid=(nblk,),
        out_shape=jax.ShapeDtypeStruct((m, n), jnp.bfloat16),
        in_specs=[pl.BlockSpec((BLK, n), lambda b: (b, 0))],
        out_specs=pl.BlockSpec((m, n), lambda b: (0, 0)),
        scratch_shapes=[
            pltpu.VMEM((1, n), jnp.float32),
            pltpu.VMEM((N_DEV - 1, 1, n), jnp.float32),
            pltpu.VMEM((1, n), jnp.float32),
            pltpu.SemaphoreType.DMA((N_DEV - 1,)),
            pltpu.SemaphoreType.DMA((N_DEV - 1,)),
        ],
        compiler_params=pltpu.CompilerParams(collective_id=0),
    )(x)
